# baseline (speedup 1.0000x reference)
import jax
import jax.numpy as jnp
from jax import lax
from jax.experimental import pallas as pl
from jax.experimental.pallas import tpu as pltpu
from jax.experimental.pallas import tpu_sc as plsc

NC = 2
NS = 16
NW = NC * NS
B = 16384
D = 32
L = 16
BPW = B // NW
V = 1000000
WIN = 1536
NFULL = V // WIN
VP = NFULL * WIN
TLEN = V - VP
NSEM = 8

_vector_mesh = plsc.VectorSubcoreMesh(
    core_axis_name="c", subcore_axis_name="s", num_cores=NC, num_subcores=NS
)


def _fire_out(flat_hbm, buf_v, sems, start):
    copies = []
    for d in range(D):
        copies.append(pltpu.async_copy(
            buf_v.at[d],
            flat_hbm.at[pl.ds(d * VP + start, WIN)],
            sems.at[d % NSEM],
        ))
    return copies


def _relayout_body(
    utabT_hbm, itabT_hbm, fu_hbm, fi_hbm, ub0, ib0, sems, su0, si0
):
    wid = lax.axis_index("s") * NC + lax.axis_index("c")

    @pl.loop(wid * WIN, VP, step=NW * WIN)
    def _(s0):
        cu0 = pltpu.async_copy(utabT_hbm.at[:, pl.ds(s0, WIN)], ub0, su0)
        ci0 = pltpu.async_copy(itabT_hbm.at[:, pl.ds(s0, WIN)], ib0, si0)
        cu0.wait()
        outs = _fire_out(fu_hbm, ub0, sems, s0)
        ci0.wait()
        outs += _fire_out(fi_hbm, ib0, sems, s0)
        for c in outs:
            c.wait()


def _gmf_body(
    fu_hbm, fi_hbm, tu_hbm, ti_hbm, uidx_hbm, iidx_hbm, wb_hbm, bb_hbm,
    out_hbm,
    uidx_v, iidx_v, ucl_v, icl_v, tu_v, ti_v, ug_v, ig_v, w_v, b_v, out_v,
    sems,
):
    wid = lax.axis_index("s") * NC + lax.axis_index("c")
    base = wid * BPW
    pltpu.sync_copy(uidx_hbm.at[pl.ds(base, BPW)], uidx_v)
    pltpu.sync_copy(iidx_hbm.at[pl.ds(base, BPW)], iidx_v)
    pltpu.sync_copy(tu_hbm, tu_v)
    pltpu.sync_copy(ti_hbm, ti_v)
    pltpu.sync_copy(wb_hbm, w_v)
    pltpu.sync_copy(bb_hbm, b_v)

    @pl.loop(0, BPW, step=L)
    def _(j):
        ucl_v[pl.ds(j, L)] = jnp.minimum(uidx_v[pl.ds(j, L)], VP - 1)
        icl_v[pl.ds(j, L)] = jnp.minimum(iidx_v[pl.ds(j, L)], VP - 1)

    copies = []
    for d in range(D):
        copies.append(pltpu.async_copy(
            fu_hbm.at[pl.ds(d * VP, VP)].at[ucl_v], ug_v.at[d],
            sems.at[d % NSEM]))
        copies.append(pltpu.async_copy(
            fi_hbm.at[pl.ds(d * VP, VP)].at[icl_v], ig_v.at[d],
            sems.at[d % NSEM]))
    for c in copies:
        c.wait()

    bias = b_v[...]

    @pl.loop(0, BPW, step=L)
    def _(j):
        uvals = uidx_v[pl.ds(j, L)]
        ivals = iidx_v[pl.ds(j, L)]
        umask = uvals >= VP
        imask = ivals >= VP
        utb = jnp.clip(uvals - VP, 0, TLEN - 1)
        itb = jnp.clip(ivals - VP, 0, TLEN - 1)
        acc = bias
        for d in range(D):
            uv = ug_v[d, pl.ds(j, L)]
            iv = ig_v[d, pl.ds(j, L)]
            ut = plsc.load_gather(tu_v, [utb + d * TLEN])
            it = plsc.load_gather(ti_v, [itb + d * TLEN])
            uv = jnp.where(umask, ut, uv)
            iv = jnp.where(imask, it, iv)
            acc = acc + w_v[d] * uv * iv
        out_v[pl.ds(j, L)] = 1.0 / (1.0 + jnp.exp(-acc))

    pltpu.sync_copy(out_v, out_hbm.at[pl.ds(base, BPW)])


@jax.jit
def _gmf(utabT, itabT, tail_u, tail_i, uidx, iidx, wb, bb):
    flat = jax.ShapeDtypeStruct((D * VP,), jnp.float32)
    k1 = pl.kernel(
        _relayout_body,
        out_type=(flat, flat),
        mesh=_vector_mesh,
        scratch_types=[
            pltpu.VMEM((D, WIN), jnp.float32),
            pltpu.VMEM((D, WIN), jnp.float32),
            pltpu.SemaphoreType.DMA((NSEM,)),
            pltpu.SemaphoreType.DMA,
            pltpu.SemaphoreType.DMA,
        ],
    )
    fu, fi = k1(utabT, itabT)

    k2 = pl.kernel(
        _gmf_body,
        out_type=jax.ShapeDtypeStruct((B,), jnp.float32),
        mesh=_vector_mesh,
        scratch_types=[
            pltpu.VMEM((BPW,), jnp.int32),
            pltpu.VMEM((BPW,), jnp.int32),
            pltpu.VMEM((BPW,), jnp.int32),
            pltpu.VMEM((BPW,), jnp.int32),
            pltpu.VMEM((D * TLEN,), jnp.float32),
            pltpu.VMEM((D * TLEN,), jnp.float32),
            pltpu.VMEM((D, BPW), jnp.float32),
            pltpu.VMEM((D, BPW), jnp.float32),
            pltpu.VMEM((D, L), jnp.float32),
            pltpu.VMEM((L,), jnp.float32),
            pltpu.VMEM((BPW,), jnp.float32),
            pltpu.SemaphoreType.DMA((NSEM,)),
        ],
        compiler_params=pltpu.CompilerParams(
            use_tc_tiling_on_sc=False, needs_layout_passes=False
        ),
    )
    return k2(fu, fi, tail_u, tail_i, uidx, iidx, wb, bb)


def kernel(userinput, iteminput, user_table, item_table, W, b):
    wb = jnp.broadcast_to(W.reshape(D, 1), (D, L))
    bb = jnp.broadcast_to(b, (L,))
    tail_u = user_table.T[:, VP:].reshape(-1)
    tail_i = item_table.T[:, VP:].reshape(-1)
    return _gmf(
        user_table.T, item_table.T, tail_u, tail_i,
        userinput.astype(jnp.int32), iteminput.astype(jnp.int32),
        wb, bb,
    )
